# baseline (speedup 1.0000x reference)
import jax
import jax.numpy as jnp
from jax import lax
from jax.experimental import pallas as pl
from jax.experimental.pallas import tpu as pltpu
from jax.experimental.pallas import tpu_sc as plsc

_B, _Q, _C = 32, 900, 91
_N = _Q * _C
_NB = _N // 256
_TB = _NB * 256
_NSTRIDE = 81904
_CAP = 512
_K = 100
_IMIN = -(2 ** 31)
_IMAX = 2 ** 31 - 1


def _monokey(bits):
    return bits ^ ((bits >> 31) & jnp.int32(0x7FFFFFFF))


def _treemax(xs):
    while len(xs) > 1:
        xs = [jnp.maximum(a, b) for a, b in zip(xs[::2], xs[1::2])] + (
            [xs[-1]] if len(xs) % 2 else [])
    return xs[0]


def _sc_body(lg_hbm, bx_hbm, ts_hbm, scores_hbm, labels_hbm, obox_hbm,
             lg_v, bx_v, ts_v, hist_v, bmax_v, ckey_v, cidx_v,
             wkey_v, widx_v, score_v, label_v, obox_v):
    bb = lax.axis_index("s") * 2 + lax.axis_index("c")
    lanes = lax.iota(jnp.int32, 16)
    ones = jnp.ones((16,), jnp.int32)
    iminv = jnp.full((16,), _IMIN, jnp.int32)

    pltpu.sync_copy(lg_hbm.at[pl.ds(bb * _NSTRIDE, _N)], lg_v)
    pltpu.sync_copy(bx_hbm.at[bb], bx_v)
    pltpu.sync_copy(ts_hbm, ts_v)

    def zero_hist(j, _):
        hist_v[j] = jnp.zeros((16,), jnp.int32)
        return 0
    lax.fori_loop(0, 256, zero_hist, 0)

    def bmaxblk(blk, _):
        base = blk * 256
        m = _treemax([lg_v[pl.ds(base + 16 * u, 16)] for u in range(16)])
        bmax_v[blk] = m
        return 0
    lax.fori_loop(0, _NB, bmaxblk, 0)
    m = _treemax([lg_v[pl.ds(_TB + 16 * u, 16)] for u in range(14)]
                 + [lg_v[pl.ds(_N - 16, 16)]])
    bmax_v[_NB] = m

    def bhist(blk, _):
        ks = _monokey(lax.bitcast_convert_type(bmax_v[blk], jnp.int32))
        d = ((ks >> 20) & 0xFFF) ^ 0x800
        plsc.addupdate_scatter(hist_v, [d >> 4, d & 15], ones)
        return 0
    lax.fori_loop(0, _NB + 1, bhist, 0)

    def scond(carry):
        return carry[1] < _K

    def sbody(carry):
        row, cum = carry
        return row - 1, cum + jnp.sum(hist_v[row])
    rend, cume = lax.while_loop(scond, sbody, (jnp.int32(255), jnp.int32(0)))
    rstar = rend + 1
    s = hist_v[rstar]
    cum0 = cume - jnp.sum(s)
    srev = lax.rev(s, (0,))
    rc = plsc.cumsum(srev)
    istar = jnp.max(plsc.all_reduce_ffs(cum0 + rc >= _K))
    t20 = rstar * 16 + (15 - istar) - 0x800
    k0 = t20 << 20
    thresh_f = jnp.broadcast_to(
        lax.bitcast_convert_type(k0 ^ ((k0 >> 31) & jnp.int32(0x7FFFFFFF)),
                                 jnp.float32), (16,))

    def init_cand(j, _):
        ckey_v[j] = iminv
        return 0
    lax.fori_loop(0, _CAP // 16, init_cand, 0)

    last = jnp.full((16,), 15, jnp.int32)

    def emit_block(nwv, xs, sels, gbases):
        css = [plsc.cumsum(s.astype(jnp.int32)) for s in sels]
        for x, sel, cs, gbase in zip(xs, sels, css, gbases):
            pos = nwv + cs - 1
            ok = jnp.logical_and(sel, pos < _CAP)
            pos = jnp.where(ok, pos, 0)
            ks = _monokey(lax.bitcast_convert_type(x, jnp.int32))
            plsc.store_scatter(ckey_v, [pos >> 4, pos & 15], ks, mask=ok)
            plsc.store_scatter(cidx_v, [pos >> 4, pos & 15],
                               gbase + lanes, mask=ok)
            nwv = nwv + cs.at[last].get(mode="promise_in_bounds")
        return nwv

    def gblk(blk, nwv):
        def hitf(nv):
            base = blk * 256
            xs = [lg_v[pl.ds(base + 16 * u, 16)] for u in range(16)]
            sels = [x >= thresh_f for x in xs]
            return emit_block(nv, xs, sels,
                              [base + 16 * u for u in range(16)])
        return lax.cond(jnp.any(bmax_v[blk] >= thresh_f), hitf,
                        lambda nv: nv, nwv)
    nwv = lax.fori_loop(0, _NB, gblk, jnp.zeros((16,), jnp.int32))

    def tailf(nv):
        xs = [lg_v[pl.ds(_TB + 16 * u, 16)] for u in range(14)]
        xs.append(lg_v[pl.ds(_N - 16, 16)])
        sels = [x >= thresh_f for x in xs[:14]]
        sels.append(jnp.logical_and(xs[14] >= thresh_f, lanes >= 4))
        return emit_block(nv, xs, sels,
                          [_TB + 16 * u for u in range(14)] + [_N - 16])
    nwv = lax.cond(jnp.any(bmax_v[_NB] >= thresh_f), tailf,
                   lambda nv: nv, nwv)
    nw = jnp.max(nwv)
    ncv = (jnp.minimum(nw, _CAP) + 15) >> 4

    def init_win(j, _):
        wkey_v[j] = iminv
        widx_v[j] = jnp.zeros((16,), jnp.int32)
        return 0
    lax.fori_loop(0, 8, init_win, 0)

    lane0 = lanes == 0

    def extract(k_, _):
        def scanv(j, carry):
            kv, pv = carry
            v = ckey_v[j]
            upd = v > kv
            kv = jnp.where(upd, v, kv)
            pv = jnp.where(upd, j * 16 + lanes, pv)
            return kv, pv
        kv, pv = lax.fori_loop(
            0, ncv, scanv, (iminv, jnp.zeros((16,), jnp.int32)))
        m = jnp.max(kv)
        pbest = jnp.min(jnp.where(kv == m, pv, jnp.int32(_IMAX)))
        ph = jnp.broadcast_to(pbest >> 4, (16,))
        plo = jnp.broadcast_to(pbest & 15, (16,))
        wk = plsc.load_gather(ckey_v, [ph, plo])
        wi = plsc.load_gather(cidx_v, [ph, plo])
        plsc.store_scatter(ckey_v, [ph, plo], iminv, mask=lane0)
        kh = jnp.broadcast_to(k_ >> 4, (16,))
        kl = jnp.broadcast_to(k_ & 15, (16,))
        plsc.store_scatter(wkey_v, [kh, kl], wk, mask=lane0)
        plsc.store_scatter(widx_v, [kh, kl], wi, mask=lane0)
        return 0
    lax.fori_loop(0, _K, extract, 0)

    bbv = jnp.broadcast_to(bb, (16,))
    hf = plsc.load_gather(ts_v, [bbv, jnp.zeros((16,), jnp.int32)]
                          ).astype(jnp.float32)
    wf = plsc.load_gather(ts_v, [bbv, jnp.ones((16,), jnp.int32)]
                          ).astype(jnp.float32)
    for j in range(8):
        ks = wkey_v[j]
        logit = lax.bitcast_convert_type(_monokey(ks), jnp.float32)
        score = 1.0 / (1.0 + jnp.exp(-logit))
        idx = widx_v[j]
        q = (
            (idx.astype(jnp.float32) + 0.5) * jnp.float32(1.0 / 91.0)
        ).astype(jnp.int32)
        lab = idx - q * _C
        c0 = jnp.zeros((16,), jnp.int32)
        cx = plsc.load_gather(bx_v, [q, c0])
        cy = plsc.load_gather(bx_v, [q, c0 + 1])
        w = plsc.load_gather(bx_v, [q, c0 + 2])
        h = plsc.load_gather(bx_v, [q, c0 + 3])
        score_v[pl.ds(j * 16, 16)] = score
        label_v[pl.ds(j * 16, 16)] = lab
        gp = (j * 16 + lanes) * 4
        plsc.store_scatter(obox_v, [gp], (cx - 0.5 * w) * wf)
        plsc.store_scatter(obox_v, [gp + 1], (cy - 0.5 * h) * hf)
        plsc.store_scatter(obox_v, [gp + 2], (cx + 0.5 * w) * wf)
        plsc.store_scatter(obox_v, [gp + 3], (cy + 0.5 * h) * hf)

    pltpu.sync_copy(score_v, scores_hbm.at[bb])
    pltpu.sync_copy(label_v, labels_hbm.at[bb])
    pltpu.sync_copy(obox_v, obox_hbm.at[bb])


@jax.jit
def _postprocess_sc(lg, bx, ts):
    mesh = plsc.VectorSubcoreMesh(core_axis_name="c", subcore_axis_name="s",
                                  num_cores=2, num_subcores=16)
    f = pl.kernel(
        _sc_body,
        out_type=(
            jax.ShapeDtypeStruct((_B, 128), jnp.float32),
            jax.ShapeDtypeStruct((_B, 128), jnp.int32),
            jax.ShapeDtypeStruct((_B, 512), jnp.float32),
        ),
        mesh=mesh,
        compiler_params=pltpu.CompilerParams(needs_layout_passes=False,
                                             use_tc_tiling_on_sc=False),
        scratch_types=[
            pltpu.VMEM((_N,), jnp.float32),
            pltpu.VMEM((_Q, 4), jnp.float32),
            pltpu.VMEM((_B, 2), jnp.int32),
            pltpu.VMEM((256, 16), jnp.int32),
            pltpu.VMEM((_NB + 1, 16), jnp.float32),
            pltpu.VMEM((_CAP // 16, 16), jnp.int32),
            pltpu.VMEM((_CAP // 16, 16), jnp.int32),
            pltpu.VMEM((8, 16), jnp.int32),
            pltpu.VMEM((8, 16), jnp.int32),
            pltpu.VMEM((128,), jnp.float32),
            pltpu.VMEM((128,), jnp.int32),
            pltpu.VMEM((512,), jnp.float32),
        ],
    )
    return f(lg, bx, ts)


def kernel(pred_logits, pred_boxes, target_sizes):
    b, q, c = pred_logits.shape
    lg = jnp.pad(pred_logits.reshape(b, q * c),
                 ((0, 0), (0, _NSTRIDE - _N))).reshape(-1)
    scores, labels, obox = _postprocess_sc(lg, pred_boxes, target_sizes)
    return (scores[:, :_K], labels[:, :_K],
            obox.reshape(b, 128, 4)[:, :_K, :])


# speedup vs baseline: 1.0217x; 1.0217x over previous
import jax
import jax.numpy as jnp
from jax import lax
from jax.experimental import pallas as pl
from jax.experimental.pallas import tpu as pltpu
from jax.experimental.pallas import tpu_sc as plsc

_B, _Q, _C = 32, 900, 91
_N = _Q * _C
_NB = _N // 256
_TB = _NB * 256
_NSTRIDE = 81904
_CAP = 512
_K = 100
_IMIN = -(2 ** 31)
_IMAX = 2 ** 31 - 1


def _monokey(bits):
    return bits ^ ((bits >> 31) & jnp.int32(0x7FFFFFFF))


def _treemax(xs):
    while len(xs) > 1:
        xs = [jnp.maximum(a, b) for a, b in zip(xs[::2], xs[1::2])] + (
            [xs[-1]] if len(xs) % 2 else [])
    return xs[0]


def _sc_body(lg_hbm, bx_hbm, ts_hbm, scores_hbm, labels_hbm, obox_hbm,
             lg_v, bx_v, ts_v, hist_v, bmax_v, ckey_v, cidx_v,
             wkey_v, widx_v, score_v, label_v, obox_v):
    bb = lax.axis_index("s") * 2 + lax.axis_index("c")
    lanes = lax.iota(jnp.int32, 16)
    ones = jnp.ones((16,), jnp.int32)
    iminv = jnp.full((16,), _IMIN, jnp.int32)
    last = jnp.full((16,), 15, jnp.int32)

    pltpu.sync_copy(lg_hbm.at[pl.ds(bb * _NSTRIDE, _N)], lg_v)
    pltpu.sync_copy(bx_hbm.at[bb], bx_v)
    pltpu.sync_copy(ts_hbm, ts_v)

    def zero_hist(j, _):
        hist_v[j] = jnp.zeros((16,), jnp.int32)
        return 0
    lax.fori_loop(0, 256, zero_hist, 0)

    def bmaxblk(blk, gacc):
        base = blk * 256
        m = _treemax([lg_v[pl.ds(base + 16 * u, 16)] for u in range(16)])
        bmax_v[blk] = m
        return jnp.maximum(gacc, m)
    gvec = lax.fori_loop(0, _NB, bmaxblk,
                         jnp.full((16,), -jnp.inf, jnp.float32))
    m = _treemax([lg_v[pl.ds(_TB + 16 * u, 16)] for u in range(14)]
                 + [lg_v[pl.ds(_N - 16, 16)]])
    bmax_v[_NB] = m
    gvec = jnp.maximum(gvec, m)

    def bhist(blk, _):
        ks = _monokey(lax.bitcast_convert_type(bmax_v[blk], jnp.int32))
        d = ((ks >> 20) & 0xFFF) ^ 0x800
        plsc.addupdate_scatter(hist_v, [d >> 4, d & 15], ones)
        return 0
    lax.fori_loop(0, _NB + 1, bhist, 0)

    gk = plsc.cummax(
        _monokey(lax.bitcast_convert_type(gvec, jnp.int32))
    ).at[last].get(mode="promise_in_bounds")
    row0 = jnp.max((((gk >> 20) & 0xFFF) ^ 0x800) >> 4)

    def scond(carry):
        return carry[1] < _K

    def sbody(carry):
        row, cum = carry
        return row - 1, cum + jnp.sum(hist_v[row])
    rend, cume = lax.while_loop(scond, sbody, (row0, jnp.int32(0)))
    rstar = rend + 1
    s = hist_v[rstar]
    cum0 = cume - jnp.sum(s)
    srev = lax.rev(s, (0,))
    rc = plsc.cumsum(srev)
    istar = jnp.max(plsc.all_reduce_ffs(cum0 + rc >= _K))
    t20 = rstar * 16 + (15 - istar) - 0x800
    k0 = t20 << 20
    thresh_f = jnp.broadcast_to(
        lax.bitcast_convert_type(k0 ^ ((k0 >> 31) & jnp.int32(0x7FFFFFFF)),
                                 jnp.float32), (16,))

    def init_cand(j, _):
        ckey_v[j] = iminv
        return 0
    lax.fori_loop(0, _CAP // 16, init_cand, 0)

    def emit_block(nwv, xs, sels, gbases):
        css = [plsc.cumsum(s.astype(jnp.int32)) for s in sels]
        for x, sel, cs, gbase in zip(xs, sels, css, gbases):
            pos = nwv + cs - 1
            ok = jnp.logical_and(sel, pos < _CAP)
            pos = jnp.where(ok, pos, 0)
            ks = _monokey(lax.bitcast_convert_type(x, jnp.int32))
            plsc.store_scatter(ckey_v, [pos >> 4, pos & 15], ks, mask=ok)
            plsc.store_scatter(cidx_v, [pos >> 4, pos & 15],
                               gbase + lanes, mask=ok)
            nwv = nwv + cs.at[last].get(mode="promise_in_bounds")
        return nwv

    def gblk(blk, nwv):
        def hitf(nv):
            base = blk * 256
            xs = [lg_v[pl.ds(base + 16 * u, 16)] for u in range(16)]
            sels = [x >= thresh_f for x in xs]
            return emit_block(nv, xs, sels,
                              [base + 16 * u for u in range(16)])
        return lax.cond(jnp.any(bmax_v[blk] >= thresh_f), hitf,
                        lambda nv: nv, nwv)
    nwv = lax.fori_loop(0, _NB, gblk, jnp.zeros((16,), jnp.int32))

    def tailf(nv):
        xs = [lg_v[pl.ds(_TB + 16 * u, 16)] for u in range(14)]
        xs.append(lg_v[pl.ds(_N - 16, 16)])
        sels = [x >= thresh_f for x in xs[:14]]
        sels.append(jnp.logical_and(xs[14] >= thresh_f, lanes >= 4))
        return emit_block(nv, xs, sels,
                          [_TB + 16 * u for u in range(14)] + [_N - 16])
    nwv = lax.cond(jnp.any(bmax_v[_NB] >= thresh_f), tailf,
                   lambda nv: nv, nwv)
    nw = jnp.max(nwv)
    ncv = (jnp.minimum(nw, _CAP) + 15) >> 4

    def init_win(j, _):
        wkey_v[j] = iminv
        widx_v[j] = jnp.zeros((16,), jnp.int32)
        return 0
    lax.fori_loop(0, 8, init_win, 0)

    lane0 = lanes == 0

    def extract(k_, _):
        def scanv(j, carry):
            kv, pv = carry
            v = ckey_v[j]
            upd = v > kv
            kv = jnp.where(upd, v, kv)
            pv = jnp.where(upd, j * 16 + lanes, pv)
            return kv, pv
        kv, pv = lax.fori_loop(
            0, ncv, scanv, (iminv, jnp.zeros((16,), jnp.int32)))
        m = plsc.cummax(kv).at[last].get(mode="promise_in_bounds")
        pm = jnp.where(kv == m, pv, jnp.int32(_IMAX))
        pbest = -plsc.cummax(-pm).at[last].get(mode="promise_in_bounds")
        ph = pbest >> 4
        plo = pbest & 15
        wk = plsc.load_gather(ckey_v, [ph, plo])
        wi = plsc.load_gather(cidx_v, [ph, plo])
        plsc.store_scatter(ckey_v, [ph, plo], iminv, mask=lane0)
        kh = jnp.broadcast_to(k_ >> 4, (16,))
        kl = jnp.broadcast_to(k_ & 15, (16,))
        plsc.store_scatter(wkey_v, [kh, kl], wk, mask=lane0)
        plsc.store_scatter(widx_v, [kh, kl], wi, mask=lane0)
        return 0
    lax.fori_loop(0, _K, extract, 0)

    bbv = jnp.broadcast_to(bb, (16,))
    hf = plsc.load_gather(ts_v, [bbv, jnp.zeros((16,), jnp.int32)]
                          ).astype(jnp.float32)
    wf = plsc.load_gather(ts_v, [bbv, jnp.ones((16,), jnp.int32)]
                          ).astype(jnp.float32)
    for j in range(8):
        ks = wkey_v[j]
        logit = lax.bitcast_convert_type(_monokey(ks), jnp.float32)
        score = 1.0 / (1.0 + jnp.exp(-logit))
        idx = widx_v[j]
        q = (
            (idx.astype(jnp.float32) + 0.5) * jnp.float32(1.0 / 91.0)
        ).astype(jnp.int32)
        lab = idx - q * _C
        c0 = jnp.zeros((16,), jnp.int32)
        cx = plsc.load_gather(bx_v, [q, c0])
        cy = plsc.load_gather(bx_v, [q, c0 + 1])
        w = plsc.load_gather(bx_v, [q, c0 + 2])
        h = plsc.load_gather(bx_v, [q, c0 + 3])
        score_v[pl.ds(j * 16, 16)] = score
        label_v[pl.ds(j * 16, 16)] = lab
        gp = (j * 16 + lanes) * 4
        plsc.store_scatter(obox_v, [gp], (cx - 0.5 * w) * wf)
        plsc.store_scatter(obox_v, [gp + 1], (cy - 0.5 * h) * hf)
        plsc.store_scatter(obox_v, [gp + 2], (cx + 0.5 * w) * wf)
        plsc.store_scatter(obox_v, [gp + 3], (cy + 0.5 * h) * hf)

    pltpu.sync_copy(score_v, scores_hbm.at[bb])
    pltpu.sync_copy(label_v, labels_hbm.at[bb])
    pltpu.sync_copy(obox_v, obox_hbm.at[bb])


@jax.jit
def _postprocess_sc(lg, bx, ts):
    mesh = plsc.VectorSubcoreMesh(core_axis_name="c", subcore_axis_name="s",
                                  num_cores=2, num_subcores=16)
    f = pl.kernel(
        _sc_body,
        out_type=(
            jax.ShapeDtypeStruct((_B, 128), jnp.float32),
            jax.ShapeDtypeStruct((_B, 128), jnp.int32),
            jax.ShapeDtypeStruct((_B, 512), jnp.float32),
        ),
        mesh=mesh,
        compiler_params=pltpu.CompilerParams(needs_layout_passes=False,
                                             use_tc_tiling_on_sc=False),
        scratch_types=[
            pltpu.VMEM((_N,), jnp.float32),
            pltpu.VMEM((_Q, 4), jnp.float32),
            pltpu.VMEM((_B, 2), jnp.int32),
            pltpu.VMEM((256, 16), jnp.int32),
            pltpu.VMEM((_NB + 1, 16), jnp.float32),
            pltpu.VMEM((_CAP // 16, 16), jnp.int32),
            pltpu.VMEM((_CAP // 16, 16), jnp.int32),
            pltpu.VMEM((8, 16), jnp.int32),
            pltpu.VMEM((8, 16), jnp.int32),
            pltpu.VMEM((128,), jnp.float32),
            pltpu.VMEM((128,), jnp.int32),
            pltpu.VMEM((512,), jnp.float32),
        ],
    )
    return f(lg, bx, ts)


def kernel(pred_logits, pred_boxes, target_sizes):
    b, q, c = pred_logits.shape
    lg = jnp.pad(pred_logits.reshape(b, q * c),
                 ((0, 0), (0, _NSTRIDE - _N))).reshape(-1)
    scores, labels, obox = _postprocess_sc(lg, pred_boxes, target_sizes)
    return (scores[:, :_K], labels[:, :_K],
            obox.reshape(b, 128, 4)[:, :_K, :])


# speedup vs baseline: 1.0259x; 1.0040x over previous
import jax
import jax.numpy as jnp
from jax import lax
from jax.experimental import pallas as pl
from jax.experimental.pallas import tpu as pltpu
from jax.experimental.pallas import tpu_sc as plsc

_B, _Q, _C = 32, 900, 91
_N = _Q * _C
_NB = _N // 256
_TB = _NB * 256
_NSTRIDE = 81904
_CAP = 512
_K = 100
_IMIN = -(2 ** 31)
_IMAX = 2 ** 31 - 1


def _monokey(bits):
    return bits ^ ((bits >> 31) & jnp.int32(0x7FFFFFFF))


def _treemax(xs):
    while len(xs) > 1:
        xs = [jnp.maximum(a, b) for a, b in zip(xs[::2], xs[1::2])] + (
            [xs[-1]] if len(xs) % 2 else [])
    return xs[0]


def _sc_body(lg_hbm, bx_hbm, ts_hbm, scores_hbm, labels_hbm, obox_hbm,
             lg_v, bx_v, ts_v, hist_v, bmax_v, ckey_v, cidx_v,
             wkey_v, widx_v, score_v, label_v, obox_v):
    bb = lax.axis_index("s") * 2 + lax.axis_index("c")
    lanes = lax.iota(jnp.int32, 16)
    ones = jnp.ones((16,), jnp.int32)
    iminv = jnp.full((16,), _IMIN, jnp.int32)
    last = jnp.full((16,), 15, jnp.int32)

    pltpu.sync_copy(lg_hbm.at[pl.ds(bb * _NSTRIDE, _N)], lg_v)
    pltpu.sync_copy(bx_hbm.at[bb], bx_v)
    pltpu.sync_copy(ts_hbm, ts_v)

    def zero_hist(j, _):
        for u in range(8):
            hist_v[j * 8 + u] = jnp.zeros((16,), jnp.int32)
        return 0
    lax.fori_loop(0, 32, zero_hist, 0)

    def bmaxblk(blk, gacc):
        base = blk * 256
        m = _treemax([lg_v[pl.ds(base + 16 * u, 16)] for u in range(16)])
        bmax_v[blk] = m
        return jnp.maximum(gacc, m)
    gvec = lax.fori_loop(0, _NB, bmaxblk,
                         jnp.full((16,), -jnp.inf, jnp.float32))
    m = _treemax([lg_v[pl.ds(_TB + 16 * u, 16)] for u in range(14)]
                 + [lg_v[pl.ds(_N - 16, 16)]])
    bmax_v[_NB] = m
    gvec = jnp.maximum(gvec, m)

    def bhist(blk, _):
        for u in range(4):
            ks = _monokey(lax.bitcast_convert_type(bmax_v[blk * 4 + u],
                                                   jnp.int32))
            d = ((ks >> 20) & 0xFFF) ^ 0x800
            plsc.addupdate_scatter(hist_v, [d >> 4, d & 15], ones)
        return 0
    lax.fori_loop(0, (_NB + 1) // 4, bhist, 0)

    gk = plsc.cummax(
        _monokey(lax.bitcast_convert_type(gvec, jnp.int32))
    ).at[last].get(mode="promise_in_bounds")
    row0 = jnp.max((((gk >> 20) & 0xFFF) ^ 0x800) >> 4)

    def scond(carry):
        return carry[1] < _K

    def sbody(carry):
        row, cum = carry
        return row - 1, cum + jnp.sum(hist_v[row])
    rend, cume = lax.while_loop(scond, sbody, (row0, jnp.int32(0)))
    rstar = rend + 1
    s = hist_v[rstar]
    cum0 = cume - jnp.sum(s)
    srev = lax.rev(s, (0,))
    rc = plsc.cumsum(srev)
    istar = jnp.max(plsc.all_reduce_ffs(cum0 + rc >= _K))
    t20 = rstar * 16 + (15 - istar) - 0x800
    k0 = t20 << 20
    thresh_f = jnp.broadcast_to(
        lax.bitcast_convert_type(k0 ^ ((k0 >> 31) & jnp.int32(0x7FFFFFFF)),
                                 jnp.float32), (16,))

    def init_cand(j, _):
        ckey_v[j] = iminv
        return 0
    lax.fori_loop(0, _CAP // 16, init_cand, 0)

    def emit_block(nwv, xs, sels, gbases):
        css = [plsc.cumsum(s.astype(jnp.int32)) for s in sels]
        for x, sel, cs, gbase in zip(xs, sels, css, gbases):
            pos = nwv + cs - 1
            ok = jnp.logical_and(sel, pos < _CAP)
            pos = jnp.where(ok, pos, 0)
            ks = _monokey(lax.bitcast_convert_type(x, jnp.int32))
            plsc.store_scatter(ckey_v, [pos >> 4, pos & 15], ks, mask=ok)
            plsc.store_scatter(cidx_v, [pos >> 4, pos & 15],
                               gbase + lanes, mask=ok)
            nwv = nwv + cs.at[last].get(mode="promise_in_bounds")
        return nwv

    def gblk(blk, nwv):
        def hitf(nv):
            base = blk * 256
            xs = [lg_v[pl.ds(base + 16 * u, 16)] for u in range(16)]
            sels = [x >= thresh_f for x in xs]
            return emit_block(nv, xs, sels,
                              [base + 16 * u for u in range(16)])
        return lax.cond(jnp.any(bmax_v[blk] >= thresh_f), hitf,
                        lambda nv: nv, nwv)
    nwv = lax.fori_loop(0, _NB, gblk, jnp.zeros((16,), jnp.int32))

    def tailf(nv):
        xs = [lg_v[pl.ds(_TB + 16 * u, 16)] for u in range(14)]
        xs.append(lg_v[pl.ds(_N - 16, 16)])
        sels = [x >= thresh_f for x in xs[:14]]
        sels.append(jnp.logical_and(xs[14] >= thresh_f, lanes >= 4))
        return emit_block(nv, xs, sels,
                          [_TB + 16 * u for u in range(14)] + [_N - 16])
    nwv = lax.cond(jnp.any(bmax_v[_NB] >= thresh_f), tailf,
                   lambda nv: nv, nwv)
    nw = jnp.max(nwv)
    ncv = (jnp.minimum(nw, _CAP) + 15) >> 4

    def init_win(j, _):
        wkey_v[j] = iminv
        widx_v[j] = jnp.zeros((16,), jnp.int32)
        return 0
    lax.fori_loop(0, 8, init_win, 0)

    lane0 = lanes == 0

    def extract(k_, _):
        def scanv(j, carry):
            kv, pv = carry
            v = ckey_v[j]
            upd = v > kv
            kv = jnp.where(upd, v, kv)
            pv = jnp.where(upd, j * 16 + lanes, pv)
            return kv, pv
        kv, pv = lax.fori_loop(
            0, ncv, scanv, (iminv, jnp.zeros((16,), jnp.int32)))
        m = plsc.cummax(kv).at[last].get(mode="promise_in_bounds")
        pm = jnp.where(kv == m, pv, jnp.int32(_IMAX))
        pbest = -plsc.cummax(-pm).at[last].get(mode="promise_in_bounds")
        ph = pbest >> 4
        plo = pbest & 15
        wk = plsc.load_gather(ckey_v, [ph, plo])
        wi = plsc.load_gather(cidx_v, [ph, plo])
        plsc.store_scatter(ckey_v, [ph, plo], iminv, mask=lane0)
        kh = jnp.broadcast_to(k_ >> 4, (16,))
        kl = jnp.broadcast_to(k_ & 15, (16,))
        plsc.store_scatter(wkey_v, [kh, kl], wk, mask=lane0)
        plsc.store_scatter(widx_v, [kh, kl], wi, mask=lane0)
        return 0
    lax.fori_loop(0, _K, extract, 0)

    bbv = jnp.broadcast_to(bb, (16,))
    hf = plsc.load_gather(ts_v, [bbv, jnp.zeros((16,), jnp.int32)]
                          ).astype(jnp.float32)
    wf = plsc.load_gather(ts_v, [bbv, jnp.ones((16,), jnp.int32)]
                          ).astype(jnp.float32)
    for j in range(8):
        ks = wkey_v[j]
        logit = lax.bitcast_convert_type(_monokey(ks), jnp.float32)
        score = 1.0 / (1.0 + jnp.exp(-logit))
        idx = widx_v[j]
        q = (
            (idx.astype(jnp.float32) + 0.5) * jnp.float32(1.0 / 91.0)
        ).astype(jnp.int32)
        lab = idx - q * _C
        c0 = jnp.zeros((16,), jnp.int32)
        cx = plsc.load_gather(bx_v, [q, c0])
        cy = plsc.load_gather(bx_v, [q, c0 + 1])
        w = plsc.load_gather(bx_v, [q, c0 + 2])
        h = plsc.load_gather(bx_v, [q, c0 + 3])
        score_v[pl.ds(j * 16, 16)] = score
        label_v[pl.ds(j * 16, 16)] = lab
        gp = (j * 16 + lanes) * 4
        plsc.store_scatter(obox_v, [gp], (cx - 0.5 * w) * wf)
        plsc.store_scatter(obox_v, [gp + 1], (cy - 0.5 * h) * hf)
        plsc.store_scatter(obox_v, [gp + 2], (cx + 0.5 * w) * wf)
        plsc.store_scatter(obox_v, [gp + 3], (cy + 0.5 * h) * hf)

    pltpu.sync_copy(score_v, scores_hbm.at[bb])
    pltpu.sync_copy(label_v, labels_hbm.at[bb])
    pltpu.sync_copy(obox_v, obox_hbm.at[bb])


@jax.jit
def _postprocess_sc(lg, bx, ts):
    mesh = plsc.VectorSubcoreMesh(core_axis_name="c", subcore_axis_name="s",
                                  num_cores=2, num_subcores=16)
    f = pl.kernel(
        _sc_body,
        out_type=(
            jax.ShapeDtypeStruct((_B, 128), jnp.float32),
            jax.ShapeDtypeStruct((_B, 128), jnp.int32),
            jax.ShapeDtypeStruct((_B, 512), jnp.float32),
        ),
        mesh=mesh,
        compiler_params=pltpu.CompilerParams(needs_layout_passes=False,
                                             use_tc_tiling_on_sc=False),
        scratch_types=[
            pltpu.VMEM((_N,), jnp.float32),
            pltpu.VMEM((_Q, 4), jnp.float32),
            pltpu.VMEM((_B, 2), jnp.int32),
            pltpu.VMEM((256, 16), jnp.int32),
            pltpu.VMEM((_NB + 1, 16), jnp.float32),
            pltpu.VMEM((_CAP // 16, 16), jnp.int32),
            pltpu.VMEM((_CAP // 16, 16), jnp.int32),
            pltpu.VMEM((8, 16), jnp.int32),
            pltpu.VMEM((8, 16), jnp.int32),
            pltpu.VMEM((128,), jnp.float32),
            pltpu.VMEM((128,), jnp.int32),
            pltpu.VMEM((512,), jnp.float32),
        ],
    )
    return f(lg, bx, ts)


def kernel(pred_logits, pred_boxes, target_sizes):
    b, q, c = pred_logits.shape
    lg = jnp.pad(pred_logits.reshape(b, q * c),
                 ((0, 0), (0, _NSTRIDE - _N))).reshape(-1)
    scores, labels, obox = _postprocess_sc(lg, pred_boxes, target_sizes)
    return (scores[:, :_K], labels[:, :_K],
            obox.reshape(b, 128, 4)[:, :_K, :])
